# baseline (speedup 1.0000x reference)
import functools

import jax
import jax.numpy as jnp
from jax import lax
from jax.experimental import pallas as pl
from jax.experimental.pallas import tpu as pltpu
from jax.experimental.pallas import tpu_sc as plsc


def _score_rank_kernel(x_ref, p_ref, b_ref, r_ref, s_ref):
    p_full = p_ref[...]
    bias = b_ref[0, 0]
    i_iota = jax.lax.broadcasted_iota(jnp.int32, (64, 512), 1)
    jlt = []
    for c in range(8):
        j_iota = jax.lax.broadcasted_iota(jnp.int32, (64, 512), 0) + (64 * c)
        jlt.append((j_iota < i_iota).astype(jnp.int32))

    for t in range(x_ref.shape[0]):
        x = x_ref[t]
        y = jax.lax.dot_general(
            x.astype(jnp.bfloat16), p_full.astype(jnp.bfloat16),
            (((1,), (0,)), ((), ())),
            preferred_element_type=jnp.float32)
        wcol = y[:, 0:1]
        s_row = jax.nn.sigmoid(wcol.T + bias)

        bits = jax.lax.bitcast_convert_type(s_row, jnp.int32)
        u_row = bits + bits
        u_col = u_row.T
        acc8 = jnp.zeros((8, 512), jnp.int32)
        for c in range(8):
            u_c = jax.lax.slice(u_col, (64 * c, 0), (64 * c + 64, 1))
            cmp = ((u_c + jlt[c]) > u_row).astype(jnp.int32)
            for r in range(8):
                acc8 = acc8 + jax.lax.slice(cmp, (8 * r, 0), (8 * r + 8, 512))
        rank_row = jnp.sum(acc8, axis=0, keepdims=True)

        r_ref[pl.ds(t, 1), :] = rank_row
        s_ref[pl.ds(t, 1), :] = s_row


def _sc_select_gather(rank_hbm, score_hbm, node_hbm,
                      gath_hbm, val_hbm,
                      rank_v, score_v, idx_v, vals_v, rows_v, sem, gsem):
    nc = 2
    wid = lax.axis_index("s") * nc + lax.axis_index("c")
    base = wid * 4
    for t in range(4):
        pltpu.async_copy(rank_hbm.at[base + t],
                         rank_v.at[t], sem)
        pltpu.async_copy(score_hbm.at[base + t],
                         score_v.at[t], sem)
    for t in range(4):
        pltpu.make_async_copy(rank_hbm.at[base + t], rank_v.at[t], sem).wait()
        pltpu.make_async_copy(score_hbm.at[base + t], score_v.at[t],
                              sem).wait()
    for t in range(4):
        for c in range(32):
            r16 = rank_v[t, pl.ds(c * 16, 16)]
            s16 = score_v[t, pl.ds(c * 16, 16)]
            i16 = lax.iota(jnp.int32, 16) + (c * 16)
            msk = r16 < 128
            plsc.store_scatter(idx_v.at[t], [r16], i16, mask=msk)
            plsc.store_scatter(vals_v, [r16 + (t * 128)], s16, mask=msk)
        pltpu.async_copy(node_hbm.at[base + t].at[idx_v.at[t]],
                         rows_v.at[t], gsem)
    for t in range(4):
        pltpu.sync_copy(vals_v.at[pl.ds(t * 128, 128)], val_hbm.at[base + t])
    for t in range(4):
        pltpu.make_async_copy(node_hbm.at[base + t].at[idx_v.at[t]],
                              rows_v.at[t], gsem).wait()
        pltpu.sync_copy(rows_v.at[t], gath_hbm.at[base + t])


def _scale_kernel(g_ref, v_ref, o_ref):
    o_ref[...] = g_ref[...] * v_ref[...][None, :, :]


@jax.jit
def kernel(node_fts, rel_edges, W, b):
    del rel_edges
    B, N, D = node_fts.shape
    K = 128
    b2 = b.reshape(1, 1).astype(jnp.float32)
    p = jnp.pad(W.reshape(D, 1), ((0, 0), (0, D - 1)))

    TB = 32
    ranks, scores = pl.pallas_call(
        _score_rank_kernel,
        grid=(B // TB,),
        in_specs=[
            pl.BlockSpec((TB, N, D), lambda i: (i, 0, 0)),
            pl.BlockSpec((D, D), lambda i: (0, 0)),
            pl.BlockSpec((1, 1), lambda i: (0, 0)),
        ],
        out_specs=[
            pl.BlockSpec((TB, N), lambda i: (i, 0)),
            pl.BlockSpec((TB, N), lambda i: (i, 0)),
        ],
        out_shape=[
            jax.ShapeDtypeStruct((B, N), jnp.int32),
            jax.ShapeDtypeStruct((B, N), jnp.float32),
        ],
    )(node_fts, p, b2)

    sc_fn = functools.partial(
        pl.kernel,
        mesh=plsc.VectorSubcoreMesh(core_axis_name="c", subcore_axis_name="s"),
        out_type=[
            jax.ShapeDtypeStruct((B, K, D), jnp.float32),
            jax.ShapeDtypeStruct((B, K), jnp.float32),
        ],
        scratch_types=[
            pltpu.VMEM((4, N), jnp.int32),
            pltpu.VMEM((4, N), jnp.float32),
            pltpu.VMEM((4, K), jnp.int32),
            pltpu.VMEM((4 * K,), jnp.float32),
            pltpu.VMEM((4, K, D), jnp.float32),
            pltpu.SemaphoreType.DMA,
            pltpu.SemaphoreType.DMA,
        ],
        compiler_params=pltpu.CompilerParams(needs_layout_passes=False),
    )(_sc_select_gather)
    gathered, value = sc_fn(ranks, scores, node_fts)

    SB = 32
    out = pl.pallas_call(
        _scale_kernel,
        grid=(B // SB,),
        in_specs=[
            pl.BlockSpec((SB, K, D), lambda i: (i, 0, 0)),
            pl.BlockSpec((K, D), lambda i: (0, 0)),
        ],
        out_specs=pl.BlockSpec((SB, K, D), lambda i: (i, 0, 0)),
        out_shape=jax.ShapeDtypeStruct((B, K, D), jnp.float32),
        compiler_params=pltpu.CompilerParams(
            dimension_semantics=("parallel",)),
    )(gathered, value)
    return out


# speedup vs baseline: 1.0074x; 1.0074x over previous
import functools

import jax
import jax.numpy as jnp
from jax import lax
from jax.experimental import pallas as pl
from jax.experimental.pallas import tpu as pltpu
from jax.experimental.pallas import tpu_sc as plsc


def _score_rank_kernel(x_ref, p_ref, b_ref, r_ref, s_ref):
    p_full = p_ref[...]
    bias = b_ref[0, 0]
    i_iota = jax.lax.broadcasted_iota(jnp.int32, (64, 512), 1)
    jlt = []
    for c in range(8):
        j_iota = jax.lax.broadcasted_iota(jnp.int32, (64, 512), 0) + (64 * c)
        jlt.append((j_iota < i_iota).astype(jnp.int32))

    for t in range(x_ref.shape[0]):
        x = x_ref[t]
        y = jax.lax.dot_general(
            x.astype(jnp.bfloat16), p_full.astype(jnp.bfloat16),
            (((1,), (0,)), ((), ())),
            preferred_element_type=jnp.float32)
        wcol = y[:, 0:1]
        s_row = jax.nn.sigmoid(wcol.T + bias)

        bits = jax.lax.bitcast_convert_type(s_row, jnp.int32)
        u_row = bits + bits
        u_col = u_row.T
        acc8 = jnp.zeros((8, 512), jnp.int32)
        for c in range(8):
            u_c = jax.lax.slice(u_col, (64 * c, 0), (64 * c + 64, 1))
            cmp = ((u_c + jlt[c]) > u_row).astype(jnp.int32)
            for r in range(8):
                acc8 = acc8 + jax.lax.slice(cmp, (8 * r, 0), (8 * r + 8, 512))
        rank_row = jnp.sum(acc8, axis=0, keepdims=True)

        r_ref[pl.ds(t, 1), :] = rank_row
        s_ref[pl.ds(t, 1), :] = s_row


def _sc_select_gather(rank_hbm, score_hbm, node_hbm,
                      gath_hbm, val_hbm,
                      rank_v, score_v, idx_v, vals_v, rows_v, sem, gsem):
    nc = 2
    wid = lax.axis_index("s") * nc + lax.axis_index("c")
    base = wid * 4
    for t in range(4):
        pltpu.async_copy(rank_hbm.at[base + t],
                         rank_v.at[t], sem)
        pltpu.async_copy(score_hbm.at[base + t],
                         score_v.at[t], sem)
    for t in range(4):
        pltpu.make_async_copy(rank_hbm.at[base + t], rank_v.at[t], sem).wait()
        pltpu.make_async_copy(score_hbm.at[base + t], score_v.at[t],
                              sem).wait()
    for t in range(4):
        for c in range(32):
            r16 = rank_v[t, pl.ds(c * 16, 16)]
            i16 = lax.iota(jnp.int32, 16) + (c * 16)
            plsc.store_scatter(idx_v.at[t], [r16], i16, mask=r16 < 128)
        pltpu.async_copy(node_hbm.at[base + t].at[idx_v.at[t]],
                         rows_v.at[t], gsem)
    for t in range(4):
        for c in range(32):
            r16 = rank_v[t, pl.ds(c * 16, 16)]
            s16 = score_v[t, pl.ds(c * 16, 16)]
            plsc.store_scatter(vals_v, [r16 + (t * 128)], s16,
                               mask=r16 < 128)
    for t in range(4):
        pltpu.sync_copy(vals_v.at[pl.ds(t * 128, 128)], val_hbm.at[base + t])
    for t in range(4):
        pltpu.make_async_copy(node_hbm.at[base + t].at[idx_v.at[t]],
                              rows_v.at[t], gsem).wait()
        pltpu.sync_copy(rows_v.at[t], gath_hbm.at[base + t])


def _scale_kernel(g_ref, v_ref, o_ref):
    o_ref[...] = g_ref[...] * v_ref[...][None, :, :]


@jax.jit
def kernel(node_fts, rel_edges, W, b):
    del rel_edges
    B, N, D = node_fts.shape
    K = 128
    b2 = b.reshape(1, 1).astype(jnp.float32)
    p = jnp.pad(W.reshape(D, 1), ((0, 0), (0, D - 1)))

    TB = 16
    ranks, scores = pl.pallas_call(
        _score_rank_kernel,
        grid=(B // TB,),
        in_specs=[
            pl.BlockSpec((TB, N, D), lambda i: (i, 0, 0)),
            pl.BlockSpec((D, D), lambda i: (0, 0)),
            pl.BlockSpec((1, 1), lambda i: (0, 0)),
        ],
        out_specs=[
            pl.BlockSpec((TB, N), lambda i: (i, 0)),
            pl.BlockSpec((TB, N), lambda i: (i, 0)),
        ],
        out_shape=[
            jax.ShapeDtypeStruct((B, N), jnp.int32),
            jax.ShapeDtypeStruct((B, N), jnp.float32),
        ],
    )(node_fts, p, b2)

    sc_fn = functools.partial(
        pl.kernel,
        mesh=plsc.VectorSubcoreMesh(core_axis_name="c", subcore_axis_name="s"),
        out_type=[
            jax.ShapeDtypeStruct((B, K, D), jnp.float32),
            jax.ShapeDtypeStruct((B, K), jnp.float32),
        ],
        scratch_types=[
            pltpu.VMEM((4, N), jnp.int32),
            pltpu.VMEM((4, N), jnp.float32),
            pltpu.VMEM((4, K), jnp.int32),
            pltpu.VMEM((4 * K,), jnp.float32),
            pltpu.VMEM((4, K, D), jnp.float32),
            pltpu.SemaphoreType.DMA,
            pltpu.SemaphoreType.DMA,
        ],
        compiler_params=pltpu.CompilerParams(needs_layout_passes=False),
    )(_sc_select_gather)
    gathered, value = sc_fn(ranks, scores, node_fts)

    SB = 32
    out = pl.pallas_call(
        _scale_kernel,
        grid=(B // SB,),
        in_specs=[
            pl.BlockSpec((SB, K, D), lambda i: (i, 0, 0)),
            pl.BlockSpec((K, D), lambda i: (0, 0)),
        ],
        out_specs=pl.BlockSpec((SB, K, D), lambda i: (i, 0, 0)),
        out_shape=jax.ShapeDtypeStruct((B, K, D), jnp.float32),
        compiler_params=pltpu.CompilerParams(
            dimension_semantics=("parallel",)),
    )(gathered, value)
    return out


# speedup vs baseline: 1.1903x; 1.1816x over previous
import functools

import jax
import jax.numpy as jnp
from jax import lax
from jax.experimental import pallas as pl
from jax.experimental.pallas import tpu as pltpu
from jax.experimental.pallas import tpu_sc as plsc


def _score_rank_kernel(x_ref, p_ref, b_ref, r_ref, s_ref):
    p_full = p_ref[...]
    bias = b_ref[0, 0]
    i_iota = jax.lax.broadcasted_iota(jnp.int32, (64, 512), 1)
    jlt = []
    for c in range(8):
        j_iota = jax.lax.broadcasted_iota(jnp.int32, (64, 512), 0) + (64 * c)
        jlt.append((j_iota < i_iota).astype(jnp.int32))

    for t in range(x_ref.shape[0]):
        x = x_ref[t]
        y = jax.lax.dot_general(
            x.astype(jnp.bfloat16), p_full.astype(jnp.bfloat16),
            (((1,), (0,)), ((), ())),
            preferred_element_type=jnp.float32)
        wcol = y[:, 0:1]
        s_col = jax.nn.sigmoid(wcol + bias)
        s_row = s_col.T

        bits_c = jax.lax.bitcast_convert_type(s_col, jnp.int32)
        u_col = bits_c + bits_c
        bits_r = jax.lax.bitcast_convert_type(s_row, jnp.int32)
        u_row = bits_r + bits_r
        acc8 = jnp.zeros((8, 512), jnp.int32)
        for c in range(8):
            u_c = jax.lax.slice(u_col, (64 * c, 0), (64 * c + 64, 1))
            cmp = ((u_c + jlt[c]) > u_row).astype(jnp.int32)
            for r in range(8):
                acc8 = acc8 + jax.lax.slice(cmp, (8 * r, 0), (8 * r + 8, 512))
        rank_row = jnp.sum(acc8, axis=0, keepdims=True)

        r_ref[pl.ds(t, 1), :] = rank_row
        s_ref[pl.ds(t, 1), :] = s_row


def _sc_select_gather(rank_hbm, score_hbm, node_hbm,
                      gath_hbm, val_hbm,
                      rank_v, score_v, idx_v, vals_v, rows_v, sem, gsem):
    nc = 2
    wid = lax.axis_index("s") * nc + lax.axis_index("c")
    base = wid * 4
    for t in range(4):
        pltpu.async_copy(rank_hbm.at[base + t],
                         rank_v.at[t], sem)
        pltpu.async_copy(score_hbm.at[base + t],
                         score_v.at[t], sem)
    for t in range(4):
        pltpu.make_async_copy(rank_hbm.at[base + t], rank_v.at[t], sem).wait()
        pltpu.make_async_copy(score_hbm.at[base + t], score_v.at[t],
                              sem).wait()
    for t in range(4):
        for c in range(32):
            r16 = rank_v[t, pl.ds(c * 16, 16)]
            i16 = lax.iota(jnp.int32, 16) + (c * 16)
            plsc.store_scatter(idx_v.at[t], [r16], i16, mask=r16 < 128)
        pltpu.async_copy(node_hbm.at[base + t].at[idx_v.at[t]],
                         rows_v.at[t], gsem)
    for t in range(4):
        for c in range(32):
            r16 = rank_v[t, pl.ds(c * 16, 16)]
            s16 = score_v[t, pl.ds(c * 16, 16)]
            plsc.store_scatter(vals_v, [r16 + (t * 128)], s16,
                               mask=r16 < 128)
    for t in range(4):
        pltpu.sync_copy(vals_v.at[pl.ds(t * 128, 128)], val_hbm.at[base + t])
    for t in range(4):
        pltpu.make_async_copy(node_hbm.at[base + t].at[idx_v.at[t]],
                              rows_v.at[t], gsem).wait()
        pltpu.sync_copy(rows_v.at[t], gath_hbm.at[base + t])


def _scale_kernel(g_ref, v_ref, o_ref):
    o_ref[...] = g_ref[...] * v_ref[...][None, :, :]


@jax.jit
def kernel(node_fts, rel_edges, W, b):
    del rel_edges
    B, N, D = node_fts.shape
    K = 128
    b2 = b.reshape(1, 1).astype(jnp.float32)
    p = jnp.pad(W.reshape(D, 1), ((0, 0), (0, D - 1)))

    TB = 16
    ranks, scores = pl.pallas_call(
        _score_rank_kernel,
        grid=(B // TB,),
        in_specs=[
            pl.BlockSpec((TB, N, D), lambda i: (i, 0, 0)),
            pl.BlockSpec((D, D), lambda i: (0, 0)),
            pl.BlockSpec((1, 1), lambda i: (0, 0)),
        ],
        out_specs=[
            pl.BlockSpec((TB, N), lambda i: (i, 0)),
            pl.BlockSpec((TB, N), lambda i: (i, 0)),
        ],
        out_shape=[
            jax.ShapeDtypeStruct((B, N), jnp.int32),
            jax.ShapeDtypeStruct((B, N), jnp.float32),
        ],
    )(node_fts, p, b2)

    sc_fn = functools.partial(
        pl.kernel,
        mesh=plsc.VectorSubcoreMesh(core_axis_name="c", subcore_axis_name="s"),
        out_type=[
            jax.ShapeDtypeStruct((B, K, D), jnp.float32),
            jax.ShapeDtypeStruct((B, K), jnp.float32),
        ],
        scratch_types=[
            pltpu.VMEM((4, N), jnp.int32),
            pltpu.VMEM((4, N), jnp.float32),
            pltpu.VMEM((4, K), jnp.int32),
            pltpu.VMEM((4 * K,), jnp.float32),
            pltpu.VMEM((4, K, D), jnp.float32),
            pltpu.SemaphoreType.DMA,
            pltpu.SemaphoreType.DMA,
        ],
        compiler_params=pltpu.CompilerParams(needs_layout_passes=False),
    )(_sc_select_gather)
    gathered, value = sc_fn(ranks, scores, node_fts)

    SB = 32
    out = pl.pallas_call(
        _scale_kernel,
        grid=(B // SB,),
        in_specs=[
            pl.BlockSpec((SB, K, D), lambda i: (i, 0, 0)),
            pl.BlockSpec((K, D), lambda i: (0, 0)),
        ],
        out_specs=pl.BlockSpec((SB, K, D), lambda i: (i, 0, 0)),
        out_shape=jax.ShapeDtypeStruct((B, K, D), jnp.float32),
        compiler_params=pltpu.CompilerParams(
            dimension_semantics=("parallel",)),
    )(gathered, value)
    return out


# speedup vs baseline: 1.2335x; 1.0363x over previous
import functools

import jax
import jax.numpy as jnp
from jax import lax
from jax.experimental import pallas as pl
from jax.experimental.pallas import tpu as pltpu
from jax.experimental.pallas import tpu_sc as plsc


def _score_rank_kernel(x_ref, p_ref, b_ref, r_ref, s_ref):
    p_full = p_ref[...]
    bias = b_ref[0, 0]
    i_iota = jax.lax.broadcasted_iota(jnp.int32, (64, 512), 1)
    jlt = []
    for c in range(8):
        j_iota = jax.lax.broadcasted_iota(jnp.int32, (64, 512), 0) + (64 * c)
        jlt.append((j_iota < i_iota).astype(jnp.int32))

    q_full = p_full.T

    for t in range(x_ref.shape[0]):
        x = x_ref[t]
        xb = x.astype(jnp.bfloat16)
        y = jax.lax.dot_general(
            xb, p_full.astype(jnp.bfloat16),
            (((1,), (0,)), ((), ())),
            preferred_element_type=jnp.float32)
        wcol = y[:, 0:1]
        s_col = jax.nn.sigmoid(wcol + bias)
        w8 = jax.lax.dot_general(
            q_full.astype(jnp.bfloat16), xb.T,
            (((1,), (0,)), ((), ())),
            preferred_element_type=jnp.float32)
        s_row = jax.nn.sigmoid(w8[0:1, :] + bias)

        bits_c = jax.lax.bitcast_convert_type(s_col, jnp.int32)
        u_col = bits_c + bits_c
        bits_r = jax.lax.bitcast_convert_type(s_row, jnp.int32)
        u_row = bits_r + bits_r
        acc8 = jnp.zeros((8, 512), jnp.int32)
        for c in range(8):
            u_c = jax.lax.slice(u_col, (64 * c, 0), (64 * c + 64, 1))
            cmp = ((u_c + jlt[c]) > u_row).astype(jnp.int32)
            for r in range(8):
                acc8 = acc8 + jax.lax.slice(cmp, (8 * r, 0), (8 * r + 8, 512))
        rank_row = jnp.sum(acc8, axis=0, keepdims=True)

        r_ref[pl.ds(t, 1), :] = rank_row
        s_ref[pl.ds(t, 1), :] = s_row


def _sc_select_gather(rank_hbm, score_hbm, node_hbm,
                      gath_hbm, val_hbm,
                      rank_v, score_v, idx_v, vals_v, rows_v, sem, gsem):
    nc = 2
    wid = lax.axis_index("s") * nc + lax.axis_index("c")
    base = wid * 4
    for t in range(4):
        pltpu.async_copy(rank_hbm.at[base + t],
                         rank_v.at[t], sem)
        pltpu.async_copy(score_hbm.at[base + t],
                         score_v.at[t], sem)
    for t in range(4):
        pltpu.make_async_copy(rank_hbm.at[base + t], rank_v.at[t], sem).wait()
        pltpu.make_async_copy(score_hbm.at[base + t], score_v.at[t],
                              sem).wait()
    for t in range(4):
        for c in range(32):
            r16 = rank_v[t, pl.ds(c * 16, 16)]
            i16 = lax.iota(jnp.int32, 16) + (c * 16)
            plsc.store_scatter(idx_v.at[t], [r16], i16, mask=r16 < 128)
        pltpu.async_copy(node_hbm.at[base + t].at[idx_v.at[t]],
                         rows_v.at[t], gsem)
    for t in range(4):
        for c in range(32):
            r16 = rank_v[t, pl.ds(c * 16, 16)]
            s16 = score_v[t, pl.ds(c * 16, 16)]
            plsc.store_scatter(vals_v, [r16 + (t * 128)], s16,
                               mask=r16 < 128)
    for t in range(4):
        pltpu.sync_copy(vals_v.at[pl.ds(t * 128, 128)], val_hbm.at[base + t])
    for t in range(4):
        pltpu.make_async_copy(node_hbm.at[base + t].at[idx_v.at[t]],
                              rows_v.at[t], gsem).wait()
        pltpu.sync_copy(rows_v.at[t], gath_hbm.at[base + t])


def _scale_kernel(g_ref, v_ref, o_ref):
    o_ref[...] = g_ref[...] * v_ref[...][None, :, :]


@jax.jit
def kernel(node_fts, rel_edges, W, b):
    del rel_edges
    B, N, D = node_fts.shape
    K = 128
    b2 = b.reshape(1, 1).astype(jnp.float32)
    p = jnp.pad(W.reshape(D, 1), ((0, 0), (0, D - 1)))

    TB = 16
    ranks, scores = pl.pallas_call(
        _score_rank_kernel,
        grid=(B // TB,),
        in_specs=[
            pl.BlockSpec((TB, N, D), lambda i: (i, 0, 0)),
            pl.BlockSpec((D, D), lambda i: (0, 0)),
            pl.BlockSpec((1, 1), lambda i: (0, 0)),
        ],
        out_specs=[
            pl.BlockSpec((TB, N), lambda i: (i, 0)),
            pl.BlockSpec((TB, N), lambda i: (i, 0)),
        ],
        out_shape=[
            jax.ShapeDtypeStruct((B, N), jnp.int32),
            jax.ShapeDtypeStruct((B, N), jnp.float32),
        ],
    )(node_fts, p, b2)

    sc_fn = functools.partial(
        pl.kernel,
        mesh=plsc.VectorSubcoreMesh(core_axis_name="c", subcore_axis_name="s"),
        out_type=[
            jax.ShapeDtypeStruct((B, K, D), jnp.float32),
            jax.ShapeDtypeStruct((B, K), jnp.float32),
        ],
        scratch_types=[
            pltpu.VMEM((4, N), jnp.int32),
            pltpu.VMEM((4, N), jnp.float32),
            pltpu.VMEM((4, K), jnp.int32),
            pltpu.VMEM((4 * K,), jnp.float32),
            pltpu.VMEM((4, K, D), jnp.float32),
            pltpu.SemaphoreType.DMA,
            pltpu.SemaphoreType.DMA,
        ],
        compiler_params=pltpu.CompilerParams(needs_layout_passes=False),
    )(_sc_select_gather)
    gathered, value = sc_fn(ranks, scores, node_fts)

    SB = 32
    out = pl.pallas_call(
        _scale_kernel,
        grid=(B // SB,),
        in_specs=[
            pl.BlockSpec((SB, K, D), lambda i: (i, 0, 0)),
            pl.BlockSpec((K, D), lambda i: (0, 0)),
        ],
        out_specs=pl.BlockSpec((SB, K, D), lambda i: (i, 0, 0)),
        out_shape=jax.ShapeDtypeStruct((B, K, D), jnp.float32),
        compiler_params=pltpu.CompilerParams(
            dimension_semantics=("parallel",)),
    )(gathered, value)
    return out
